# baseline (speedup 1.0000x reference)
import functools

import jax
import jax.numpy as jnp
from jax.experimental import pallas as pl
from jax.experimental.pallas import tpu as pltpu

N = 10000
DEG = 16
F = 256
H = 128
C = 40
B = 400
NB = N // B
K = 5
EPS = 1e-5


def _k1(x_ref, nb_ref, w1xt_ref, w1nt_ref, g1_ref, b1_ref,
        x1_ref, f2_ref, s1_ref, ss1_ref):
    g1 = g1_ref[0, 0]
    b1 = b1_ref[0, 0]
    nb2d = nb_ref[...]
    w1xt = w1xt_ref[...].astype(jnp.bfloat16)
    nb1 = jnp.dot(nb2d.astype(jnp.bfloat16), w1xt,
                  preferred_element_type=jnp.float32)
    nb3 = nb1.reshape(B, DEG, H)
    m = jnp.mean(nb3, axis=(1, 2), keepdims=True)
    d = nb3 - m
    v = jnp.mean(d * d, axis=(1, 2), keepdims=True)
    y = jnp.maximum(d * jax.lax.rsqrt(v + EPS) * g1 + b1, 0.0)
    f2_ref[...] = jnp.mean(y, axis=1)
    f = jnp.mean(nb2d.reshape(B, DEG, F), axis=1)
    x1 = (jnp.dot(x_ref[...].astype(jnp.bfloat16), w1xt,
                  preferred_element_type=jnp.float32)
          + jnp.dot(f.astype(jnp.bfloat16), w1nt_ref[...].astype(jnp.bfloat16),
                    preferred_element_type=jnp.float32))
    x1_ref[...] = x1
    s1_ref[...] = jnp.sum(x1.reshape(B // 8, 8, H), axis=0)
    ss1_ref[...] = jnp.sum((x1 * x1).reshape(B // 8, 8, H), axis=0)


def _k2(x1_ref, f2_ref, s1_ref, ss1_ref, w2xt_ref, w2nt_ref, g1_ref, b1_ref,
        x2_ref, s2_ref, ss2_ref):
    cnt = float(N * H)
    m1 = jnp.sum(s1_ref[...]) / cnt
    v1 = jnp.sum(ss1_ref[...]) / cnt - m1 * m1
    g1 = g1_ref[0, 0]
    b1 = b1_ref[0, 0]
    x1 = x1_ref[...]
    x1n = jnp.maximum((x1 - m1) * jax.lax.rsqrt(v1 + EPS) * g1 + b1, 0.0)
    x2 = (jnp.dot(x1n, w2xt_ref[...], preferred_element_type=jnp.float32)
          + jnp.dot(f2_ref[...], w2nt_ref[...],
                    preferred_element_type=jnp.float32))
    x2_ref[...] = x2
    s2_ref[...] = jnp.sum(x2.reshape(B // 8, 8, H), axis=0)
    ss2_ref[...] = jnp.sum((x2 * x2).reshape(B // 8, 8, H), axis=0)


def _k3(x2_ref, s2_ref, ss2_ref, wct_ref, bc_ref, g2_ref, b2_ref, out_ref):
    cnt = float(N * H)
    m2 = jnp.sum(s2_ref[...]) / cnt
    v2 = jnp.sum(ss2_ref[...]) / cnt - m2 * m2
    g2 = g2_ref[0, 0]
    b2 = b2_ref[0, 0]
    x2 = x2_ref[...]
    x2n = jnp.maximum((x2 - m2) * jax.lax.rsqrt(v2 + EPS) * g2 + b2, 0.0)
    out_ref[...] = (jnp.dot(x2n, wct_ref[...], preferred_element_type=jnp.float32)
                    + bc_ref[...])


def _smem11():
    return pl.BlockSpec(memory_space=pltpu.SMEM)


def _full():
    return pl.BlockSpec(memory_space=pltpu.VMEM)


@functools.partial(jax.jit)
def kernel(x, neighbor, W1x, W1n, W2x, W2n, g1, b1, g2, b2, Wc, bc):
    g1s = g1.reshape(1, 1)
    b1s = b1.reshape(1, 1)
    g2s = g2.reshape(1, 1)
    b2s = b2.reshape(1, 1)

    x2d = x.reshape(N, F)
    w1xt = W1x.T
    w1nt = W1n.T

    CH = N // K
    NBC = CH // B
    x1s, f2s, s1s, ss1s = [], [], [], []
    for c in range(K):
        nbc = neighbor[c * CH:(c + 1) * CH].reshape(CH * DEG, F)
        x1c, f2c, s1c, ss1c = pl.pallas_call(
            _k1,
            grid=(NBC,),
            in_specs=[
                pl.BlockSpec((B, F), lambda i, c=c: (c * NBC + i, 0)),
                pl.BlockSpec((B * DEG, F), lambda i: (i, 0)),
                _full(),
                _full(),
                _smem11(),
                _smem11(),
            ],
            out_specs=[
                pl.BlockSpec((B, H), lambda i: (i, 0)),
                pl.BlockSpec((B, H), lambda i: (i, 0)),
                pl.BlockSpec((8, H), lambda i: (i, 0)),
                pl.BlockSpec((8, H), lambda i: (i, 0)),
            ],
            out_shape=[
                jax.ShapeDtypeStruct((CH, H), jnp.float32),
                jax.ShapeDtypeStruct((CH, H), jnp.float32),
                jax.ShapeDtypeStruct((NBC * 8, H), jnp.float32),
                jax.ShapeDtypeStruct((NBC * 8, H), jnp.float32),
            ],
            compiler_params=pltpu.CompilerParams(
                dimension_semantics=("parallel",)),
        )(x2d, nbc, w1xt, w1nt, g1s, b1s)
        x1s.append(x1c)
        f2s.append(f2c)
        s1s.append(s1c)
        ss1s.append(ss1c)
    x1 = jnp.concatenate(x1s, axis=0)
    f2 = jnp.concatenate(f2s, axis=0)
    s1 = jnp.concatenate(s1s, axis=0)
    ss1 = jnp.concatenate(ss1s, axis=0)

    x2, s2, ss2 = pl.pallas_call(
        _k2,
        grid=(NB,),
        in_specs=[
            pl.BlockSpec((B, H), lambda i: (i, 0)),
            pl.BlockSpec((B, H), lambda i: (i, 0)),
            pl.BlockSpec((NB * 8, H), lambda i: (0, 0)),
            pl.BlockSpec((NB * 8, H), lambda i: (0, 0)),
            _full(),
            _full(),
            _smem11(),
            _smem11(),
        ],
        out_specs=[
            pl.BlockSpec((B, H), lambda i: (i, 0)),
            pl.BlockSpec((8, H), lambda i: (i, 0)),
            pl.BlockSpec((8, H), lambda i: (i, 0)),
        ],
        out_shape=[
            jax.ShapeDtypeStruct((N, H), jnp.float32),
            jax.ShapeDtypeStruct((NB * 8, H), jnp.float32),
            jax.ShapeDtypeStruct((NB * 8, H), jnp.float32),
        ],
        compiler_params=pltpu.CompilerParams(
            dimension_semantics=("parallel",)),
    )(x1, f2, s1, ss1, W2x.T, W2n.T, g1s, b1s)

    out = pl.pallas_call(
        _k3,
        grid=(NB,),
        in_specs=[
            pl.BlockSpec((B, H), lambda i: (i, 0)),
            pl.BlockSpec((NB * 8, H), lambda i: (0, 0)),
            pl.BlockSpec((NB * 8, H), lambda i: (0, 0)),
            _full(),
            _full(),
            _smem11(),
            _smem11(),
        ],
        out_specs=pl.BlockSpec((B, C), lambda i: (i, 0)),
        out_shape=jax.ShapeDtypeStruct((N, C), jnp.float32),
        compiler_params=pltpu.CompilerParams(
            dimension_semantics=("parallel",)),
    )(x2, s2, ss2, Wc.T, bc.reshape(1, C), g2s, b2s)

    return out


# speedup vs baseline: 1.7083x; 1.7083x over previous
import jax
import jax.numpy as jnp
from jax.experimental import pallas as pl
from jax.experimental.pallas import tpu as pltpu

N = 10000
DEG = 16
F = 256
H = 128
C = 40
B = 400
NB = N // B
B2 = 2000
NB2 = N // B2
EPS = 1e-5
CNT = float(N * H)


def _kall(x_ref, nb_ref, w1xt_ref, w1nt_ref, w2xt_ref, w2nt_ref, wct_ref,
          bc_ref, g1_ref, b1_ref, g2_ref, b2_ref, out_ref,
          x1_scr, f2_scr, x2_scr, s1_scr, ss1_scr, s2_scr, ss2_scr):
    s = pl.program_id(0)
    g1 = g1_ref[0, 0]
    b1 = b1_ref[0, 0]

    @pl.when(s < NB)
    def _phase0():
        i = s
        nb2d = nb_ref[...]
        w1xt = w1xt_ref[...].astype(jnp.bfloat16)
        nb1 = jnp.dot(nb2d.astype(jnp.bfloat16), w1xt,
                      preferred_element_type=jnp.float32)
        nb3 = nb1.reshape(B, DEG, H)
        m = jnp.mean(nb3, axis=(1, 2), keepdims=True)
        d = nb3 - m
        v = jnp.mean(d * d, axis=(1, 2), keepdims=True)
        y = jnp.maximum(d * jax.lax.rsqrt(v + EPS) * g1 + b1, 0.0)
        f2_scr[pl.ds(i * B, B), :] = jnp.mean(y, axis=1)
        f = jnp.mean(nb2d.reshape(B, DEG, F), axis=1)
        x1 = (jnp.dot(x_ref[...].astype(jnp.bfloat16), w1xt,
                      preferred_element_type=jnp.float32)
              + jnp.dot(f.astype(jnp.bfloat16),
                        w1nt_ref[...].astype(jnp.bfloat16),
                        preferred_element_type=jnp.float32))
        x1_scr[pl.ds(i * B, B), :] = x1
        ps = jnp.sum(x1.reshape(B // 8, 8, H), axis=0)
        pss = jnp.sum((x1 * x1).reshape(B // 8, 8, H), axis=0)

        @pl.when(i == 0)
        def _():
            s1_scr[...] = ps
            ss1_scr[...] = pss

        @pl.when(i > 0)
        def _():
            s1_scr[...] += ps
            ss1_scr[...] += pss

    @pl.when((s >= NB) & (s < NB + NB2))
    def _phase1():
        j = s - NB
        m1 = jnp.sum(s1_scr[...]) / CNT
        v1 = jnp.sum(ss1_scr[...]) / CNT - m1 * m1
        x1 = x1_scr[pl.ds(j * B2, B2), :]
        x1n = jnp.maximum((x1 - m1) * jax.lax.rsqrt(v1 + EPS) * g1 + b1, 0.0)
        x2 = (jnp.dot(x1n, w2xt_ref[...], preferred_element_type=jnp.float32)
              + jnp.dot(f2_scr[pl.ds(j * B2, B2), :], w2nt_ref[...],
                        preferred_element_type=jnp.float32))
        x2_scr[pl.ds(j * B2, B2), :] = x2
        ps = jnp.sum(x2.reshape(B2 // 8, 8, H), axis=0)
        pss = jnp.sum((x2 * x2).reshape(B2 // 8, 8, H), axis=0)

        @pl.when(j == 0)
        def _():
            s2_scr[...] = ps
            ss2_scr[...] = pss

        @pl.when(j > 0)
        def _():
            s2_scr[...] += ps
            ss2_scr[...] += pss

    @pl.when(s >= NB + NB2)
    def _phase2():
        j = s - NB - NB2
        m2 = jnp.sum(s2_scr[...]) / CNT
        v2 = jnp.sum(ss2_scr[...]) / CNT - m2 * m2
        g2 = g2_ref[0, 0]
        b2 = b2_ref[0, 0]
        x2 = x2_scr[pl.ds(j * B2, B2), :]
        x2n = jnp.maximum((x2 - m2) * jax.lax.rsqrt(v2 + EPS) * g2 + b2, 0.0)
        out_ref[...] = (jnp.dot(x2n, wct_ref[...],
                                preferred_element_type=jnp.float32)
                        + bc_ref[...])


def _smem11():
    return pl.BlockSpec(memory_space=pltpu.SMEM)


def _full():
    return pl.BlockSpec(memory_space=pltpu.VMEM)


@jax.jit
def kernel(x, neighbor, W1x, W1n, W2x, W2n, g1, b1, g2, b2, Wc, bc):
    x2d = x.reshape(N, F)
    nb2d = neighbor.reshape(N * DEG, F)
    g1s = g1.reshape(1, 1)
    b1s = b1.reshape(1, 1)
    g2s = g2.reshape(1, 1)
    b2s = b2.reshape(1, 1)

    out = pl.pallas_call(
        _kall,
        grid=(NB + 2 * NB2,),
        in_specs=[
            pl.BlockSpec((B, F), lambda s: (jnp.minimum(s, NB - 1), 0)),
            pl.BlockSpec((B * DEG, F), lambda s: (jnp.minimum(s, NB - 1), 0)),
            _full(),
            _full(),
            _full(),
            _full(),
            _full(),
            _full(),
            _smem11(),
            _smem11(),
            _smem11(),
            _smem11(),
        ],
        out_specs=pl.BlockSpec(
            (B2, C), lambda s: (jnp.maximum(s - (NB + NB2), 0), 0)),
        out_shape=jax.ShapeDtypeStruct((N, C), jnp.float32),
        scratch_shapes=[
            pltpu.VMEM((N, H), jnp.float32),
            pltpu.VMEM((N, H), jnp.float32),
            pltpu.VMEM((N, H), jnp.float32),
            pltpu.VMEM((8, H), jnp.float32),
            pltpu.VMEM((8, H), jnp.float32),
            pltpu.VMEM((8, H), jnp.float32),
            pltpu.VMEM((8, H), jnp.float32),
        ],
        compiler_params=pltpu.CompilerParams(
            dimension_semantics=("arbitrary",)),
    )(x2d, nb2d, W1x.T, W1n.T, W2x.T, W2n.T, Wc.T, bc.reshape(1, C),
      g1s, b1s, g2s, b2s)

    return out
